# baseline (speedup 1.0000x reference)
import functools

import jax
import jax.numpy as jnp
from jax import lax
from jax.experimental import pallas as pl
from jax.experimental.pallas import tpu as pltpu
from jax.experimental.pallas import tpu_sc as plsc

B = 4096
CONT = 13
NFIELDS = 26
VOCAB = 100000
D = 32
SLOTS = CONT + NFIELDS

NC = 2
NS = 16
NW = NC * NS
BPW = B // NW


def _tc_cont_body(c_ref, t_ref, o_ref):
    o_ref[...] = c_ref[...][:, :, None] * t_ref[...][None, :, :]


def _tc_cont(continuous, cont_table):
    return pl.pallas_call(
        _tc_cont_body,
        out_shape=jax.ShapeDtypeStruct((B, CONT, D), jnp.float32),
    )(continuous, cont_table)


def _sc_main(cat_tables, idx_wfj, cont_embed):
    mesh = plsc.VectorSubcoreMesh(core_axis_name="c", subcore_axis_name="s")

    @functools.partial(
        pl.kernel,
        mesh=mesh,
        out_type=jax.ShapeDtypeStruct((B, SLOTS, D), jnp.float32),
        compiler_params=pltpu.CompilerParams(use_tc_tiling_on_sc=False),
        scratch_types=[
            pltpu.VMEM((NFIELDS, BPW), jnp.int32),
            pltpu.VMEM((NFIELDS * BPW, D), jnp.float32),
            pltpu.SemaphoreType.DMA,
            pltpu.SemaphoreType.DMA,
        ],
    )
    def k(tab_hbm, idx_hbm, cont_hbm, out_hbm, idx_v, rows_v, gsem, wsem):
        wid = lax.axis_index("s") * NC + lax.axis_index("c")
        b0 = wid * BPW
        pltpu.sync_copy(idx_hbm.at[wid], idx_v)
        gathers = []
        for f in range(NFIELDS):
            gathers.append(
                pltpu.async_copy(
                    tab_hbm.at[f].at[idx_v.at[f]],
                    rows_v.at[pl.ds(f * BPW, BPW)],
                    gsem,
                )
            )
        cont_write = pltpu.async_copy(
            cont_hbm.at[pl.ds(b0, BPW)],
            out_hbm.at[pl.ds(b0, BPW), pl.ds(0, CONT)],
            wsem,
        )
        for g in gathers:
            g.wait()
        writes = []
        for f in range(NFIELDS):
            writes.append(
                pltpu.async_copy(
                    rows_v.at[pl.ds(f * BPW, BPW)],
                    out_hbm.at[pl.ds(b0, BPW), CONT + f],
                    wsem,
                )
            )
        cont_write.wait()
        for w in writes:
            w.wait()

    return k(cat_tables, idx_wfj, cont_embed)


def kernel(continuous, categorical, cont_table, cat_tables):
    cont_embed = _tc_cont(continuous, cont_table)
    idx_wfj = categorical.reshape(NW, BPW, NFIELDS).transpose(0, 2, 1)
    return _sc_main(cat_tables, idx_wfj, cont_embed)


# speedup vs baseline: 1.1861x; 1.1861x over previous
import functools

import jax
import jax.numpy as jnp
from jax import lax
from jax.experimental import pallas as pl
from jax.experimental.pallas import tpu as pltpu
from jax.experimental.pallas import tpu_sc as plsc

B = 4096
CONT = 13
NFIELDS = 26
VOCAB = 100000
D = 32

NC = 2
NS = 16
NW = NC * NS

_ROWS = B * NFIELDS
_RPW = _ROWS // NW


def _sc_gather(table, flat_idx):
    mesh = plsc.VectorSubcoreMesh(core_axis_name="c", subcore_axis_name="s")

    @functools.partial(
        pl.kernel,
        mesh=mesh,
        out_type=jax.ShapeDtypeStruct((_ROWS, D), jnp.float32),
        compiler_params=pltpu.CompilerParams(use_tc_tiling_on_sc=False),
        scratch_types=[
            pltpu.VMEM((_RPW,), jnp.int32),
            pltpu.VMEM((_RPW, D), jnp.float32),
            pltpu.SemaphoreType.DMA,
        ],
    )
    def k(table_hbm, idx_hbm, out_hbm, idx_v, rows_v, sem):
        wid = lax.axis_index("s") * NC + lax.axis_index("c")
        base = wid * _RPW
        pltpu.sync_copy(idx_hbm.at[pl.ds(base, _RPW)], idx_v)
        pltpu.async_copy(table_hbm.at[idx_v], rows_v, sem).wait()
        pltpu.sync_copy(rows_v, out_hbm.at[pl.ds(base, _RPW)])

    return k(table, flat_idx)


def _tc_cont_body(c_ref, t_ref, o_ref):
    o_ref[...] = c_ref[...][:, :, None] * t_ref[...][None, :, :]


def _tc_cont(continuous, cont_table):
    return pl.pallas_call(
        _tc_cont_body,
        out_shape=jax.ShapeDtypeStruct((B, CONT, D), jnp.float32),
    )(continuous, cont_table)


def kernel(continuous, categorical, cont_table, cat_tables):
    table = cat_tables.reshape(NFIELDS * VOCAB, D)
    offsets = (jnp.arange(NFIELDS, dtype=jnp.int32) * VOCAB)[None, :]
    flat_idx = (categorical + offsets).reshape(_ROWS)
    cat_embed = _sc_gather(table, flat_idx).reshape(B, NFIELDS, D)
    cont_embed = _tc_cont(continuous, cont_table)
    return jnp.concatenate([cont_embed, cat_embed], axis=1)
